# baseline (speedup 1.0000x reference)
import functools

import jax
import jax.numpy as jnp
from jax import lax
from jax.experimental import pallas as pl
from jax.experimental.pallas import tpu as pltpu
from jax.experimental.pallas import tpu_sc as plsc

N = 10000
E = 320000
D = 128
H = 64
HW = H // 2
B = 2
L = 12
OUT = 12

BN_ROWS = 2000
LSTM_ROWS = 5000

NC = 2
NS = 16
N_PAD = 10112
SUB_ROWS = N_PAD // NS
CHUNK = 800
N_CHUNKS = 13
E_PAD = NC * NS * N_CHUNKS * CHUNK


def _tc1_body(x_ref, w1_ref, dega_ref, degb_ref, ya_ref, yb_ref, dinv_ref):
    deg = dega_ref[...] + degb_ref[...]
    dinv = jnp.where(deg > 0, lax.rsqrt(deg), 0.0)
    xw = jnp.dot(x_ref[...], w1_ref[...], preferred_element_type=jnp.float32)
    y = xw * dinv
    ya_ref[...] = y[:, :HW]
    yb_ref[...] = y[:, HW:]
    dinv_ref[...] = dinv


def _tc1(x, w1, deg_a, deg_b):
    nb = N // BN_ROWS
    return pl.pallas_call(
        _tc1_body,
        grid=(nb,),
        in_specs=[
            pl.BlockSpec((BN_ROWS, D), lambda i: (i, 0)),
            pl.BlockSpec((D, H), lambda i: (0, 0)),
            pl.BlockSpec((BN_ROWS, 1), lambda i: (i, 0)),
            pl.BlockSpec((BN_ROWS, 1), lambda i: (i, 0)),
        ],
        out_specs=[
            pl.BlockSpec((BN_ROWS, HW), lambda i: (i, 0)),
            pl.BlockSpec((BN_ROWS, HW), lambda i: (i, 0)),
            pl.BlockSpec((BN_ROWS, 1), lambda i: (i, 0)),
        ],
        out_shape=[
            jax.ShapeDtypeStruct((N, HW), jnp.float32),
            jax.ShapeDtypeStruct((N, HW), jnp.float32),
            jax.ShapeDtypeStruct((N, 1), jnp.float32),
        ],
    )(x, w1, deg_a, deg_b)


def _tc2_body(p00_ref, p01_ref, p10_ref, p11_ref, ya_ref, yb_ref,
              dinv_ref, b1_ref, w2_ref, y2a_ref, y2b_ref):
    dinv = dinv_ref[...]
    acc = jnp.concatenate(
        [p00_ref[...] + p10_ref[...], p01_ref[...] + p11_ref[...]], axis=1)
    y1 = jnp.concatenate([ya_ref[...], yb_ref[...]], axis=1)
    h1 = jnp.maximum(dinv * (acc + y1) + b1_ref[...], 0.0)
    y2 = jnp.dot(h1, w2_ref[...], preferred_element_type=jnp.float32) * dinv
    y2a_ref[...] = y2[:, :HW]
    y2b_ref[...] = y2[:, HW:]


def _tc2(p00, p01, p10, p11, y1a, y1b, dinv, b1, w2):
    nb = N // BN_ROWS
    half_spec = pl.BlockSpec((BN_ROWS, HW), lambda i: (i, 0))
    return pl.pallas_call(
        _tc2_body,
        grid=(nb,),
        in_specs=[
            half_spec, half_spec, half_spec, half_spec,
            half_spec, half_spec,
            pl.BlockSpec((BN_ROWS, 1), lambda i: (i, 0)),
            pl.BlockSpec((1, H), lambda i: (0, 0)),
            pl.BlockSpec((H, H), lambda i: (0, 0)),
        ],
        out_specs=[half_spec, half_spec],
        out_shape=[
            jax.ShapeDtypeStruct((N, HW), jnp.float32),
            jax.ShapeDtypeStruct((N, HW), jnp.float32),
        ],
    )(p00, p01, p10, p11, y1a, y1b, dinv, b1, w2)


def _tc3_body(q00_ref, q01_ref, q10_ref, q11_ref, ya_ref, yb_ref,
              dinv_ref, b2_ref, dem_ref, wih_ref, whh_ref,
              bg_ref, wo_h_ref, wo_s_ref, bo_ref, out_ref):
    dinv = dinv_ref[...]
    acc = jnp.concatenate(
        [q00_ref[...] + q10_ref[...], q01_ref[...] + q11_ref[...]], axis=1)
    y2 = jnp.concatenate([ya_ref[...], yb_ref[...]], axis=1)
    h0 = jnp.maximum(dinv * (acc + y2) + b2_ref[...], 0.0)
    dem = dem_ref[...]
    wih = wih_ref[...]
    bg = bg_ref[...]
    c0 = jnp.zeros_like(h0)

    lane = lax.broadcasted_iota(jnp.int32, (1, L), 1)

    def step(t, carry):
        h, c = carry
        x_t = jnp.sum(jnp.where(lane == t, dem, 0.0), axis=1, keepdims=True)
        gates = (
            lax.dot_general(h, whh_ref[...], (((1,), (1,)), ((), ())),
                            preferred_element_type=jnp.float32)
            + x_t * wih + bg
        )
        i = jax.nn.sigmoid(gates[:, 0 * H:1 * H])
        f = jax.nn.sigmoid(gates[:, 1 * H:2 * H])
        g = jnp.tanh(gates[:, 2 * H:3 * H])
        o = jax.nn.sigmoid(gates[:, 3 * H:4 * H])
        c = f * c + i * g
        h = o * jnp.tanh(c)
        return (h, c)

    hn, _ = lax.fori_loop(0, L, step, (h0, c0))
    pred = (
        lax.dot_general(hn, wo_h_ref[...], (((1,), (1,)), ((), ())),
                        preferred_element_type=jnp.float32)
        + lax.dot_general(h0, wo_s_ref[...], (((1,), (1,)), ((), ())),
                          preferred_element_type=jnp.float32)
        + bo_ref[...]
    )
    out_ref[...] = pred


def _tc3(q00, q01, q10, q11, y2a, y2b, dinv, b2, dem_bn,
         wih, whh, bg, wo_h, wo_s, bo):
    nb = (B * N) // LSTM_ROWS
    nb_n = N // LSTM_ROWS

    def node_blk(i):
        return (i % nb_n, 0)

    half_spec = pl.BlockSpec((LSTM_ROWS, HW), node_blk)
    return pl.pallas_call(
        _tc3_body,
        grid=(nb,),
        in_specs=[
            half_spec, half_spec, half_spec, half_spec,
            half_spec, half_spec,
            pl.BlockSpec((LSTM_ROWS, 1), node_blk),
            pl.BlockSpec((1, H), lambda i: (0, 0)),
            pl.BlockSpec((LSTM_ROWS, L), lambda i: (i, 0)),
            pl.BlockSpec((1, 4 * H), lambda i: (0, 0)),
            pl.BlockSpec((4 * H, H), lambda i: (0, 0)),
            pl.BlockSpec((1, 4 * H), lambda i: (0, 0)),
            pl.BlockSpec((OUT, H), lambda i: (0, 0)),
            pl.BlockSpec((OUT, H), lambda i: (0, 0)),
            pl.BlockSpec((1, OUT), lambda i: (0, 0)),
        ],
        out_specs=pl.BlockSpec((LSTM_ROWS, OUT), lambda i: (i, 0)),
        out_shape=jax.ShapeDtypeStruct((B * N, OUT), jnp.float32),
        compiler_params=pltpu.CompilerParams(
            vmem_limit_bytes=100 * 1024 * 1024),
    )(q00, q01, q10, q11, y2a, y2b, dinv, b2, dem_bn,
      wih, whh, bg, wo_h, wo_s, bo)


_SC_MESH = plsc.VectorSubcoreMesh(core_axis_name="c", subcore_axis_name="s")


def _sc_deg(col_p, ew_p):
    @functools.partial(
        pl.kernel,
        out_type=jax.ShapeDtypeStruct((NC * N_PAD,), jnp.float32),
        mesh=_SC_MESH,
        compiler_params=pltpu.CompilerParams(use_tc_tiling_on_sc=False),
        scratch_types=[
            pltpu.VMEM((CHUNK,), jnp.int32),
            pltpu.VMEM((CHUNK,), jnp.float32),
            pltpu.VMEM((CHUNK,), jnp.float32),
            pltpu.VMEM_SHARED((N_PAD,), jnp.float32),
            pltpu.SemaphoreType.DMA,
        ],
    )
    def k(col_hbm, ew_hbm, out_hbm, col_v, ew_v, stage_v, acc, sem):
        cid = lax.axis_index("c")
        sid = lax.axis_index("s")

        def fill(i, _):
            stage_v[pl.ds(i * 16, 16)] = jnp.full((16,), 0.5, jnp.float32)
            return 0
        lax.fori_loop(0, CHUNK // 16, fill, 0)
        pltpu.sync_copy(stage_v.at[pl.ds(0, SUB_ROWS)],
                        acc.at[pl.ds(sid * SUB_ROWS, SUB_ROWS)])
        plsc.subcore_barrier()

        half = E_PAD // NC
        per_sub = half // NS
        base0 = cid * half + sid * per_sub
        for kk in range(N_CHUNKS):
            b = base0 + kk * CHUNK
            pltpu.sync_copy(col_hbm.at[pl.ds(b, CHUNK)], col_v)
            pltpu.sync_copy(ew_hbm.at[pl.ds(b, CHUNK)], ew_v)
            pltpu.sync_copy(ew_v, acc.at[col_v], add=True)
        plsc.subcore_barrier()

        pltpu.sync_copy(acc.at[pl.ds(sid * SUB_ROWS, SUB_ROWS)],
                        stage_v.at[pl.ds(0, SUB_ROWS)])
        pltpu.sync_copy(stage_v.at[pl.ds(0, SUB_ROWS)],
                        out_hbm.at[pl.ds(cid * N_PAD + sid * SUB_ROWS, SUB_ROWS)])

    return k(col_p, ew_p)


def _sc_agg(y_a, y_b, row_p, col_p, ew_p):
    @functools.partial(
        pl.kernel,
        out_type=jax.ShapeDtypeStruct((NC * 2 * N_PAD, HW), jnp.float32),
        mesh=_SC_MESH,
        compiler_params=pltpu.CompilerParams(use_tc_tiling_on_sc=False),
        scratch_types=[
            pltpu.VMEM((CHUNK,), jnp.int32),
            pltpu.VMEM((CHUNK,), jnp.int32),
            pltpu.VMEM((CHUNK,), jnp.float32),
            pltpu.VMEM((CHUNK, HW), jnp.float32),
            pltpu.VMEM_SHARED((N_PAD, HW), jnp.float32),
            pltpu.VMEM_SHARED((N, HW), jnp.float32),
            pltpu.VMEM_SHARED((N, HW), jnp.float32),
            pltpu.SemaphoreType.DMA,
        ],
    )
    def k(ya_hbm, yb_hbm, row_hbm, col_hbm, ew_hbm, out_hbm,
          row_v, col_v, ew_v, rows_v, acc, ya_s, yb_s, sem):
        cid = lax.axis_index("c")
        sid = lax.axis_index("s")

        for src, dst in ((ya_hbm, ya_s), (yb_hbm, yb_s)):
            @pl.when(sid < NS - 1)
            def _(src=src, dst=dst):
                pltpu.sync_copy(src.at[pl.ds(sid * 640, 640)],
                                dst.at[pl.ds(sid * 640, 640)])

            @pl.when(sid == NS - 1)
            def _(src=src, dst=dst):
                pltpu.sync_copy(src.at[pl.ds(9600, N - 9600)],
                                dst.at[pl.ds(9600, N - 9600)])

        zeros16 = jnp.zeros((16,), jnp.float32)
        half_e = E_PAD // NC
        per_sub = half_e // NS
        base0 = cid * half_e + sid * per_sub

        for half, y_s in ((0, ya_s), (1, yb_s)):
            def zrow(e, _):
                for j in range(HW // 16):
                    rows_v[e, pl.ds(j * 16, 16)] = zeros16
                return 0
            lax.fori_loop(0, SUB_ROWS, zrow, 0)
            pltpu.sync_copy(rows_v.at[pl.ds(0, SUB_ROWS)],
                            acc.at[pl.ds(sid * SUB_ROWS, SUB_ROWS)])
            plsc.subcore_barrier()

            for kk in range(N_CHUNKS):
                b = base0 + kk * CHUNK
                pltpu.sync_copy(row_hbm.at[pl.ds(b, CHUNK)], row_v)
                pltpu.sync_copy(col_hbm.at[pl.ds(b, CHUNK)], col_v)
                pltpu.sync_copy(ew_hbm.at[pl.ds(b, CHUNK)], ew_v)
                pltpu.async_copy(y_s.at[row_v], rows_v, sem).wait()

                def scale(g, _):
                    w = ew_v[pl.ds(g * 16, 16)]
                    for i in range(16):
                        s = w[i]
                        e = g * 16 + i
                        for j in range(HW // 16):
                            sl = pl.ds(j * 16, 16)
                            rows_v[e, sl] = rows_v[e, sl] * s
                    return 0
                lax.fori_loop(0, CHUNK // 16, scale, 0)

                pltpu.sync_copy(rows_v, acc.at[col_v], add=True)
            plsc.subcore_barrier()

            pltpu.sync_copy(acc.at[pl.ds(sid * SUB_ROWS, SUB_ROWS)],
                            rows_v.at[pl.ds(0, SUB_ROWS)])
            pltpu.sync_copy(
                rows_v.at[pl.ds(0, SUB_ROWS)],
                out_hbm.at[pl.ds((cid * 2 + half) * N_PAD + sid * SUB_ROWS,
                                 SUB_ROWS)])

    return k(y_a, y_b, row_p, col_p, ew_p)


def kernel(x_static, edge_index, edge_attr, demand_seq,
           W_gcn1, b_gcn1, W_gcn2, b_gcn2,
           W_ih, W_hh, b_ih, b_hh, W_out, b_out):
    pad = E_PAD - E
    row_p = jnp.concatenate([edge_index[0], jnp.zeros((pad,), jnp.int32)])
    col_p = jnp.concatenate([edge_index[1], jnp.zeros((pad,), jnp.int32)])
    ew_p = jnp.concatenate([edge_attr, jnp.zeros((pad,), jnp.float32)])

    deg = _sc_deg(col_p, ew_p)
    y1a, y1b, dinv = _tc1(x_static, W_gcn1, deg[:N_PAD, None], deg[N_PAD:, None])

    acc1 = _sc_agg(y1a, y1b, row_p, col_p, ew_p)
    y2a, y2b = _tc2(acc1[0:N_PAD], acc1[N_PAD:2 * N_PAD],
                    acc1[2 * N_PAD:3 * N_PAD], acc1[3 * N_PAD:4 * N_PAD],
                    y1a, y1b, dinv, b_gcn1[None], W_gcn2)

    acc2 = _sc_agg(y2a, y2b, row_p, col_p, ew_p)

    dem_bn = jnp.transpose(demand_seq, (0, 2, 1)).reshape(B * N, L)
    pred = _tc3(acc2[0:N_PAD], acc2[N_PAD:2 * N_PAD],
                acc2[2 * N_PAD:3 * N_PAD], acc2[3 * N_PAD:4 * N_PAD],
                y2a, y2b, dinv, b_gcn2[None], dem_bn,
                W_ih.reshape(1, 4 * H), W_hh, (b_ih + b_hh)[None],
                W_out[:, :H], W_out[:, H:], b_out[None])
    return pred.reshape(B, N, OUT)


# speedup vs baseline: 1.0792x; 1.0792x over previous
import functools

import jax
import jax.numpy as jnp
from jax import lax
from jax.experimental import pallas as pl
from jax.experimental.pallas import tpu as pltpu
from jax.experimental.pallas import tpu_sc as plsc

N = 10000
E = 320000
D = 128
H = 64
HW = H // 2
B = 2
L = 12
OUT = 12

BN_ROWS = 2000
LSTM_ROWS = 2000

NC = 2
NS = 16
N_PAD = 10112
SUB_ROWS = N_PAD // NS
CHUNK = 800
N_CHUNKS = 13
E_PAD = NC * NS * N_CHUNKS * CHUNK


def _tc1_body(x_ref, w1_ref, dega_ref, degb_ref, ya_ref, yb_ref, dinv_ref):
    deg = dega_ref[...] + degb_ref[...]
    dinv = jnp.where(deg > 0, lax.rsqrt(deg), 0.0)
    xw = jnp.dot(x_ref[...], w1_ref[...], preferred_element_type=jnp.float32)
    y = xw * dinv
    ya_ref[...] = y[:, :HW]
    yb_ref[...] = y[:, HW:]
    dinv_ref[...] = dinv


def _tc1(x, w1, deg_a, deg_b):
    nb = N // BN_ROWS
    return pl.pallas_call(
        _tc1_body,
        grid=(nb,),
        in_specs=[
            pl.BlockSpec((BN_ROWS, D), lambda i: (i, 0)),
            pl.BlockSpec((D, H), lambda i: (0, 0)),
            pl.BlockSpec((BN_ROWS, 1), lambda i: (i, 0)),
            pl.BlockSpec((BN_ROWS, 1), lambda i: (i, 0)),
        ],
        out_specs=[
            pl.BlockSpec((BN_ROWS, HW), lambda i: (i, 0)),
            pl.BlockSpec((BN_ROWS, HW), lambda i: (i, 0)),
            pl.BlockSpec((BN_ROWS, 1), lambda i: (i, 0)),
        ],
        out_shape=[
            jax.ShapeDtypeStruct((N, HW), jnp.float32),
            jax.ShapeDtypeStruct((N, HW), jnp.float32),
            jax.ShapeDtypeStruct((N, 1), jnp.float32),
        ],
    )(x, w1, deg_a, deg_b)


def _tc2_body(p00_ref, p01_ref, p10_ref, p11_ref, ya_ref, yb_ref,
              dinv_ref, b1_ref, w2_ref, y2a_ref, y2b_ref):
    dinv = dinv_ref[...]
    acc = jnp.concatenate(
        [p00_ref[...] + p10_ref[...], p01_ref[...] + p11_ref[...]], axis=1)
    y1 = jnp.concatenate([ya_ref[...], yb_ref[...]], axis=1)
    h1 = jnp.maximum(dinv * (acc + y1) + b1_ref[...], 0.0)
    y2 = jnp.dot(h1, w2_ref[...], preferred_element_type=jnp.float32) * dinv
    y2a_ref[...] = y2[:, :HW]
    y2b_ref[...] = y2[:, HW:]


def _tc2(p00, p01, p10, p11, y1a, y1b, dinv, b1, w2):
    nb = N // BN_ROWS
    half_spec = pl.BlockSpec((BN_ROWS, HW), lambda i: (i, 0))
    return pl.pallas_call(
        _tc2_body,
        grid=(nb,),
        in_specs=[
            half_spec, half_spec, half_spec, half_spec,
            half_spec, half_spec,
            pl.BlockSpec((BN_ROWS, 1), lambda i: (i, 0)),
            pl.BlockSpec((1, H), lambda i: (0, 0)),
            pl.BlockSpec((H, H), lambda i: (0, 0)),
        ],
        out_specs=[half_spec, half_spec],
        out_shape=[
            jax.ShapeDtypeStruct((N, HW), jnp.float32),
            jax.ShapeDtypeStruct((N, HW), jnp.float32),
        ],
    )(p00, p01, p10, p11, y1a, y1b, dinv, b1, w2)


def _tc3_body(q00_ref, q01_ref, q10_ref, q11_ref, ya_ref, yb_ref,
              dinv_ref, b2_ref, dem_ref, wih_ref, whh_ref,
              bg_ref, wo_h_ref, wo_s_ref, bo_ref, out_ref):
    dinv = dinv_ref[...]
    acc = jnp.concatenate(
        [q00_ref[...] + q10_ref[...], q01_ref[...] + q11_ref[...]], axis=1)
    y2 = jnp.concatenate([ya_ref[...], yb_ref[...]], axis=1)
    h0 = jnp.maximum(dinv * (acc + y2) + b2_ref[...], 0.0)
    dem = dem_ref[...]
    wih = wih_ref[...]
    bg = bg_ref[...]
    whh = whh_ref[...]

    def sigm(x):
        return 0.5 * jnp.tanh(0.5 * x) + 0.5

    h = h0
    c = jnp.zeros_like(h0)
    for t in range(L):
        x_t = dem[:, t:t + 1]
        gates = (
            lax.dot_general(h, whh, (((1,), (1,)), ((), ())),
                            preferred_element_type=jnp.float32)
            + x_t * wih + bg
        )
        i = sigm(gates[:, 0 * H:1 * H])
        f = sigm(gates[:, 1 * H:2 * H])
        g = jnp.tanh(gates[:, 2 * H:3 * H])
        o = sigm(gates[:, 3 * H:4 * H])
        c = f * c + i * g
        h = o * jnp.tanh(c)
    hn = h
    pred = (
        lax.dot_general(hn, wo_h_ref[...], (((1,), (1,)), ((), ())),
                        preferred_element_type=jnp.float32)
        + lax.dot_general(h0, wo_s_ref[...], (((1,), (1,)), ((), ())),
                          preferred_element_type=jnp.float32)
        + bo_ref[...]
    )
    out_ref[...] = pred


def _tc3(q00, q01, q10, q11, y2a, y2b, dinv, b2, dem_bn,
         wih, whh, bg, wo_h, wo_s, bo):
    nb = (B * N) // LSTM_ROWS
    nb_n = N // LSTM_ROWS

    def node_blk(i):
        return (i % nb_n, 0)

    half_spec = pl.BlockSpec((LSTM_ROWS, HW), node_blk)
    return pl.pallas_call(
        _tc3_body,
        grid=(nb,),
        in_specs=[
            half_spec, half_spec, half_spec, half_spec,
            half_spec, half_spec,
            pl.BlockSpec((LSTM_ROWS, 1), node_blk),
            pl.BlockSpec((1, H), lambda i: (0, 0)),
            pl.BlockSpec((LSTM_ROWS, L), lambda i: (i, 0)),
            pl.BlockSpec((1, 4 * H), lambda i: (0, 0)),
            pl.BlockSpec((4 * H, H), lambda i: (0, 0)),
            pl.BlockSpec((1, 4 * H), lambda i: (0, 0)),
            pl.BlockSpec((OUT, H), lambda i: (0, 0)),
            pl.BlockSpec((OUT, H), lambda i: (0, 0)),
            pl.BlockSpec((1, OUT), lambda i: (0, 0)),
        ],
        out_specs=pl.BlockSpec((LSTM_ROWS, OUT), lambda i: (i, 0)),
        out_shape=jax.ShapeDtypeStruct((B * N, OUT), jnp.float32),
        compiler_params=pltpu.CompilerParams(
            vmem_limit_bytes=100 * 1024 * 1024),
    )(q00, q01, q10, q11, y2a, y2b, dinv, b2, dem_bn,
      wih, whh, bg, wo_h, wo_s, bo)


_SC_MESH = plsc.VectorSubcoreMesh(core_axis_name="c", subcore_axis_name="s")


def _sc_deg(col_p, ew_p):
    @functools.partial(
        pl.kernel,
        out_type=jax.ShapeDtypeStruct((NC * N_PAD,), jnp.float32),
        mesh=_SC_MESH,
        compiler_params=pltpu.CompilerParams(use_tc_tiling_on_sc=False),
        scratch_types=[
            pltpu.VMEM((CHUNK,), jnp.int32),
            pltpu.VMEM((CHUNK,), jnp.float32),
            pltpu.VMEM((CHUNK,), jnp.float32),
            pltpu.VMEM_SHARED((N_PAD,), jnp.float32),
            pltpu.SemaphoreType.DMA,
        ],
    )
    def k(col_hbm, ew_hbm, out_hbm, col_v, ew_v, stage_v, acc, sem):
        cid = lax.axis_index("c")
        sid = lax.axis_index("s")

        def fill(i, _):
            stage_v[pl.ds(i * 16, 16)] = jnp.full((16,), 0.5, jnp.float32)
            return 0
        lax.fori_loop(0, CHUNK // 16, fill, 0)
        pltpu.sync_copy(stage_v.at[pl.ds(0, SUB_ROWS)],
                        acc.at[pl.ds(sid * SUB_ROWS, SUB_ROWS)])
        plsc.subcore_barrier()

        half = E_PAD // NC
        per_sub = half // NS
        base0 = cid * half + sid * per_sub
        for kk in range(N_CHUNKS):
            b = base0 + kk * CHUNK
            pltpu.sync_copy(col_hbm.at[pl.ds(b, CHUNK)], col_v)
            pltpu.sync_copy(ew_hbm.at[pl.ds(b, CHUNK)], ew_v)
            pltpu.sync_copy(ew_v, acc.at[col_v], add=True)
        plsc.subcore_barrier()

        pltpu.sync_copy(acc.at[pl.ds(sid * SUB_ROWS, SUB_ROWS)],
                        stage_v.at[pl.ds(0, SUB_ROWS)])
        pltpu.sync_copy(stage_v.at[pl.ds(0, SUB_ROWS)],
                        out_hbm.at[pl.ds(cid * N_PAD + sid * SUB_ROWS, SUB_ROWS)])

    return k(col_p, ew_p)


def _sc_agg(y_a, y_b, row_p, col_p, ew_p):
    @functools.partial(
        pl.kernel,
        out_type=jax.ShapeDtypeStruct((NC * 2 * N_PAD, HW), jnp.float32),
        mesh=_SC_MESH,
        compiler_params=pltpu.CompilerParams(use_tc_tiling_on_sc=False),
        scratch_types=[
            pltpu.VMEM((CHUNK,), jnp.int32),
            pltpu.VMEM((CHUNK,), jnp.int32),
            pltpu.VMEM((CHUNK,), jnp.float32),
            pltpu.VMEM((CHUNK, HW), jnp.float32),
            pltpu.VMEM_SHARED((N_PAD, HW), jnp.float32),
            pltpu.VMEM_SHARED((N, HW), jnp.float32),
            pltpu.VMEM_SHARED((N, HW), jnp.float32),
            pltpu.SemaphoreType.DMA,
        ],
    )
    def k(ya_hbm, yb_hbm, row_hbm, col_hbm, ew_hbm, out_hbm,
          row_v, col_v, ew_v, rows_v, acc, ya_s, yb_s, sem):
        cid = lax.axis_index("c")
        sid = lax.axis_index("s")

        for src, dst in ((ya_hbm, ya_s), (yb_hbm, yb_s)):
            @pl.when(sid < NS - 1)
            def _(src=src, dst=dst):
                pltpu.sync_copy(src.at[pl.ds(sid * 640, 640)],
                                dst.at[pl.ds(sid * 640, 640)])

            @pl.when(sid == NS - 1)
            def _(src=src, dst=dst):
                pltpu.sync_copy(src.at[pl.ds(9600, N - 9600)],
                                dst.at[pl.ds(9600, N - 9600)])

        zeros16 = jnp.zeros((16,), jnp.float32)
        half_e = E_PAD // NC
        per_sub = half_e // NS
        base0 = cid * half_e + sid * per_sub

        for half, y_s in ((0, ya_s), (1, yb_s)):
            def zrow(e, _):
                for j in range(HW // 16):
                    rows_v[e, pl.ds(j * 16, 16)] = zeros16
                return 0
            lax.fori_loop(0, SUB_ROWS, zrow, 0)
            pltpu.sync_copy(rows_v.at[pl.ds(0, SUB_ROWS)],
                            acc.at[pl.ds(sid * SUB_ROWS, SUB_ROWS)])
            plsc.subcore_barrier()

            for kk in range(N_CHUNKS):
                b = base0 + kk * CHUNK
                pltpu.sync_copy(row_hbm.at[pl.ds(b, CHUNK)], row_v)
                pltpu.sync_copy(col_hbm.at[pl.ds(b, CHUNK)], col_v)
                pltpu.sync_copy(ew_hbm.at[pl.ds(b, CHUNK)], ew_v)
                pltpu.async_copy(y_s.at[row_v], rows_v, sem).wait()

                def scale(g, _):
                    w = ew_v[pl.ds(g * 16, 16)]
                    for i in range(16):
                        s = w[i]
                        e = g * 16 + i
                        for j in range(HW // 16):
                            sl = pl.ds(j * 16, 16)
                            rows_v[e, sl] = rows_v[e, sl] * s
                    return 0
                lax.fori_loop(0, CHUNK // 16, scale, 0)

                pltpu.sync_copy(rows_v, acc.at[col_v], add=True)
            plsc.subcore_barrier()

            pltpu.sync_copy(acc.at[pl.ds(sid * SUB_ROWS, SUB_ROWS)],
                            rows_v.at[pl.ds(0, SUB_ROWS)])
            pltpu.sync_copy(
                rows_v.at[pl.ds(0, SUB_ROWS)],
                out_hbm.at[pl.ds((cid * 2 + half) * N_PAD + sid * SUB_ROWS,
                                 SUB_ROWS)])

    return k(y_a, y_b, row_p, col_p, ew_p)


def kernel(x_static, edge_index, edge_attr, demand_seq,
           W_gcn1, b_gcn1, W_gcn2, b_gcn2,
           W_ih, W_hh, b_ih, b_hh, W_out, b_out):
    pad = E_PAD - E
    row_p = jnp.concatenate([edge_index[0], jnp.zeros((pad,), jnp.int32)])
    col_p = jnp.concatenate([edge_index[1], jnp.zeros((pad,), jnp.int32)])
    ew_p = jnp.concatenate([edge_attr, jnp.zeros((pad,), jnp.float32)])

    deg = _sc_deg(col_p, ew_p)
    y1a, y1b, dinv = _tc1(x_static, W_gcn1, deg[:N_PAD, None], deg[N_PAD:, None])

    acc1 = _sc_agg(y1a, y1b, row_p, col_p, ew_p)
    y2a, y2b = _tc2(acc1[0:N_PAD], acc1[N_PAD:2 * N_PAD],
                    acc1[2 * N_PAD:3 * N_PAD], acc1[3 * N_PAD:4 * N_PAD],
                    y1a, y1b, dinv, b_gcn1[None], W_gcn2)

    acc2 = _sc_agg(y2a, y2b, row_p, col_p, ew_p)

    dem_bn = jnp.transpose(demand_seq, (0, 2, 1)).reshape(B * N, L)
    pred = _tc3(acc2[0:N_PAD], acc2[N_PAD:2 * N_PAD],
                acc2[2 * N_PAD:3 * N_PAD], acc2[3 * N_PAD:4 * N_PAD],
                y2a, y2b, dinv, b_gcn2[None], dem_bn,
                W_ih.reshape(1, 4 * H), W_hh, (b_ih + b_hh)[None],
                W_out[:, :H], W_out[:, H:], b_out[None])
    return pred.reshape(B, N, OUT)


# speedup vs baseline: 1.2382x; 1.1473x over previous
import functools

import jax
import jax.numpy as jnp
from jax import lax
from jax.experimental import pallas as pl
from jax.experimental.pallas import tpu as pltpu
from jax.experimental.pallas import tpu_sc as plsc

N = 10000
E = 320000
D = 128
H = 64
HW = H // 2
B = 2
L = 12
OUT = 12

BN_ROWS = 2000
LSTM_ROWS = 2000

NC = 2
NS = 16
N_PAD = 10112
SUB_ROWS = N_PAD // NS
CHUNK = 800
N_CHUNKS = 13
E_PAD = NC * NS * N_CHUNKS * CHUNK


def _tc1_body(x_ref, w1_ref, dega_ref, degb_ref, ya_ref, yb_ref, dinv_ref):
    deg = dega_ref[...] + degb_ref[...]
    dinv = jnp.where(deg > 0, lax.rsqrt(deg), 0.0)
    xw = jnp.dot(x_ref[...], w1_ref[...], preferred_element_type=jnp.float32)
    y = xw * dinv
    ya_ref[...] = y[:, :HW]
    yb_ref[...] = y[:, HW:]
    dinv_ref[...] = dinv


def _tc1(x, w1, deg_a, deg_b):
    nb = N // BN_ROWS
    return pl.pallas_call(
        _tc1_body,
        grid=(nb,),
        in_specs=[
            pl.BlockSpec((BN_ROWS, D), lambda i: (i, 0)),
            pl.BlockSpec((D, H), lambda i: (0, 0)),
            pl.BlockSpec((BN_ROWS, 1), lambda i: (i, 0)),
            pl.BlockSpec((BN_ROWS, 1), lambda i: (i, 0)),
        ],
        out_specs=[
            pl.BlockSpec((BN_ROWS, HW), lambda i: (i, 0)),
            pl.BlockSpec((BN_ROWS, HW), lambda i: (i, 0)),
            pl.BlockSpec((BN_ROWS, 1), lambda i: (i, 0)),
        ],
        out_shape=[
            jax.ShapeDtypeStruct((N, HW), jnp.float32),
            jax.ShapeDtypeStruct((N, HW), jnp.float32),
            jax.ShapeDtypeStruct((N, 1), jnp.float32),
        ],
    )(x, w1, deg_a, deg_b)


def _tc2_body(p00_ref, p01_ref, p10_ref, p11_ref, ya_ref, yb_ref,
              dinv_ref, b1_ref, w2_ref, y2a_ref, y2b_ref):
    dinv = dinv_ref[...]
    acc = jnp.concatenate(
        [p00_ref[...] + p10_ref[...], p01_ref[...] + p11_ref[...]], axis=1)
    y1 = jnp.concatenate([ya_ref[...], yb_ref[...]], axis=1)
    h1 = jnp.maximum(dinv * (acc + y1) + b1_ref[...], 0.0)
    y2 = jnp.dot(h1, w2_ref[...], preferred_element_type=jnp.float32) * dinv
    y2a_ref[...] = y2[:, :HW]
    y2b_ref[...] = y2[:, HW:]


def _tc2(p00, p01, p10, p11, y1a, y1b, dinv, b1, w2):
    nb = N // BN_ROWS
    half_spec = pl.BlockSpec((BN_ROWS, HW), lambda i: (i, 0))
    return pl.pallas_call(
        _tc2_body,
        grid=(nb,),
        in_specs=[
            half_spec, half_spec, half_spec, half_spec,
            half_spec, half_spec,
            pl.BlockSpec((BN_ROWS, 1), lambda i: (i, 0)),
            pl.BlockSpec((1, H), lambda i: (0, 0)),
            pl.BlockSpec((H, H), lambda i: (0, 0)),
        ],
        out_specs=[half_spec, half_spec],
        out_shape=[
            jax.ShapeDtypeStruct((N, HW), jnp.float32),
            jax.ShapeDtypeStruct((N, HW), jnp.float32),
        ],
    )(p00, p01, p10, p11, y1a, y1b, dinv, b1, w2)


def _tc3_body(q00_ref, q01_ref, q10_ref, q11_ref, ya_ref, yb_ref,
              dinv_ref, b2_ref, dem_ref, wih_ref, whh_ref,
              bg_ref, wo_h_ref, wo_s_ref, bo_ref, out_ref):
    dinv = dinv_ref[...]
    acc = jnp.concatenate(
        [q00_ref[...] + q10_ref[...], q01_ref[...] + q11_ref[...]], axis=1)
    y2 = jnp.concatenate([ya_ref[...], yb_ref[...]], axis=1)
    h0 = jnp.maximum(dinv * (acc + y2) + b2_ref[...], 0.0)
    dem = dem_ref[...]
    wih = wih_ref[...]
    bg = bg_ref[...]
    whh = whh_ref[...]

    def sigm(x):
        return 0.5 * jnp.tanh(0.5 * x) + 0.5

    h = h0
    c = jnp.zeros_like(h0)
    for t in range(L):
        x_t = dem[:, t:t + 1]
        gates = (
            lax.dot_general(h, whh, (((1,), (1,)), ((), ())),
                            preferred_element_type=jnp.float32)
            + x_t * wih + bg
        )
        i = sigm(gates[:, 0 * H:1 * H])
        f = sigm(gates[:, 1 * H:2 * H])
        g = jnp.tanh(gates[:, 2 * H:3 * H])
        o = sigm(gates[:, 3 * H:4 * H])
        c = f * c + i * g
        h = o * jnp.tanh(c)
    hn = h
    pred = (
        lax.dot_general(hn, wo_h_ref[...], (((1,), (1,)), ((), ())),
                        preferred_element_type=jnp.float32)
        + lax.dot_general(h0, wo_s_ref[...], (((1,), (1,)), ((), ())),
                          preferred_element_type=jnp.float32)
        + bo_ref[...]
    )
    out_ref[...] = pred


def _tc3(q00, q01, q10, q11, y2a, y2b, dinv, b2, dem_bn,
         wih, whh, bg, wo_h, wo_s, bo):
    nb = (B * N) // LSTM_ROWS
    nb_n = N // LSTM_ROWS

    def node_blk(i):
        return (i % nb_n, 0)

    half_spec = pl.BlockSpec((LSTM_ROWS, HW), node_blk)
    return pl.pallas_call(
        _tc3_body,
        grid=(nb,),
        in_specs=[
            half_spec, half_spec, half_spec, half_spec,
            half_spec, half_spec,
            pl.BlockSpec((LSTM_ROWS, 1), node_blk),
            pl.BlockSpec((1, H), lambda i: (0, 0)),
            pl.BlockSpec((LSTM_ROWS, L), lambda i: (i, 0)),
            pl.BlockSpec((1, 4 * H), lambda i: (0, 0)),
            pl.BlockSpec((4 * H, H), lambda i: (0, 0)),
            pl.BlockSpec((1, 4 * H), lambda i: (0, 0)),
            pl.BlockSpec((OUT, H), lambda i: (0, 0)),
            pl.BlockSpec((OUT, H), lambda i: (0, 0)),
            pl.BlockSpec((1, OUT), lambda i: (0, 0)),
        ],
        out_specs=pl.BlockSpec((LSTM_ROWS, OUT), lambda i: (i, 0)),
        out_shape=jax.ShapeDtypeStruct((B * N, OUT), jnp.float32),
        compiler_params=pltpu.CompilerParams(
            vmem_limit_bytes=100 * 1024 * 1024),
    )(q00, q01, q10, q11, y2a, y2b, dinv, b2, dem_bn,
      wih, whh, bg, wo_h, wo_s, bo)


_SC_MESH = plsc.VectorSubcoreMesh(core_axis_name="c", subcore_axis_name="s")


def _sc_deg(col_p, ew_p):
    @functools.partial(
        pl.kernel,
        out_type=jax.ShapeDtypeStruct((NC * N_PAD,), jnp.float32),
        mesh=_SC_MESH,
        compiler_params=pltpu.CompilerParams(use_tc_tiling_on_sc=False),
        scratch_types=[
            pltpu.VMEM((CHUNK,), jnp.int32),
            pltpu.VMEM((CHUNK,), jnp.float32),
            pltpu.VMEM((CHUNK,), jnp.float32),
            pltpu.VMEM_SHARED((N_PAD,), jnp.float32),
            pltpu.SemaphoreType.DMA,
        ],
    )
    def k(col_hbm, ew_hbm, out_hbm, col_v, ew_v, stage_v, acc, sem):
        cid = lax.axis_index("c")
        sid = lax.axis_index("s")

        def fill(i, _):
            stage_v[pl.ds(i * 16, 16)] = jnp.full((16,), 0.5, jnp.float32)
            return 0
        lax.fori_loop(0, CHUNK // 16, fill, 0)
        pltpu.sync_copy(stage_v.at[pl.ds(0, SUB_ROWS)],
                        acc.at[pl.ds(sid * SUB_ROWS, SUB_ROWS)])
        plsc.subcore_barrier()

        half = E_PAD // NC
        per_sub = half // NS
        base0 = cid * half + sid * per_sub
        for kk in range(N_CHUNKS):
            b = base0 + kk * CHUNK
            pltpu.sync_copy(col_hbm.at[pl.ds(b, CHUNK)], col_v)
            pltpu.sync_copy(ew_hbm.at[pl.ds(b, CHUNK)], ew_v)
            pltpu.sync_copy(ew_v, acc.at[col_v], add=True)
        plsc.subcore_barrier()

        pltpu.sync_copy(acc.at[pl.ds(sid * SUB_ROWS, SUB_ROWS)],
                        stage_v.at[pl.ds(0, SUB_ROWS)])
        pltpu.sync_copy(stage_v.at[pl.ds(0, SUB_ROWS)],
                        out_hbm.at[pl.ds(cid * N_PAD + sid * SUB_ROWS, SUB_ROWS)])

    return k(col_p, ew_p)


def _sc_agg(y_a, y_b, row_p, col_p, ew_p):
    @functools.partial(
        pl.kernel,
        out_type=jax.ShapeDtypeStruct((NC * 2 * N_PAD, HW), jnp.float32),
        mesh=_SC_MESH,
        compiler_params=pltpu.CompilerParams(use_tc_tiling_on_sc=False),
        scratch_types=[
            pltpu.VMEM((CHUNK,), jnp.int32),
            pltpu.VMEM((CHUNK,), jnp.int32),
            pltpu.VMEM((CHUNK,), jnp.int32),
            pltpu.VMEM((CHUNK,), jnp.int32),
            pltpu.VMEM((CHUNK,), jnp.float32),
            pltpu.VMEM((CHUNK,), jnp.float32),
            pltpu.VMEM((CHUNK, HW), jnp.float32),
            pltpu.VMEM((CHUNK, HW), jnp.float32),
            pltpu.VMEM_SHARED((N_PAD, HW), jnp.float32),
            pltpu.VMEM_SHARED((N, HW), jnp.float32),
            pltpu.VMEM_SHARED((N, HW), jnp.float32),
            pltpu.SemaphoreType.DMA,
            pltpu.SemaphoreType.DMA,
            pltpu.SemaphoreType.DMA,
            pltpu.SemaphoreType.DMA,
            pltpu.SemaphoreType.DMA,
            pltpu.SemaphoreType.DMA,
        ],
    )
    def k(ya_hbm, yb_hbm, row_hbm, col_hbm, ew_hbm, out_hbm,
          row_v0, row_v1, col_v0, col_v1, ew_v0, ew_v1, rows_v0, rows_v1,
          acc, ya_s, yb_s, gsem0, gsem1, ssem0, ssem1, isem0, isem1):
        cid = lax.axis_index("c")
        sid = lax.axis_index("s")
        bufs = ((row_v0, col_v0, ew_v0, rows_v0, gsem0, ssem0, isem0),
                (row_v1, col_v1, ew_v1, rows_v1, gsem1, ssem1, isem1))

        for src, dst in ((ya_hbm, ya_s), (yb_hbm, yb_s)):
            @pl.when(sid < NS - 1)
            def _(src=src, dst=dst):
                pltpu.sync_copy(src.at[pl.ds(sid * 640, 640)],
                                dst.at[pl.ds(sid * 640, 640)])

            @pl.when(sid == NS - 1)
            def _(src=src, dst=dst):
                pltpu.sync_copy(src.at[pl.ds(9600, N - 9600)],
                                dst.at[pl.ds(9600, N - 9600)])

        zeros16 = jnp.zeros((16,), jnp.float32)
        half_e = E_PAD // NC
        per_sub = half_e // NS
        base0 = cid * half_e + sid * per_sub

        def idx_load(bi, b):
            row_v, col_v, ew_v, _, _, _, isem = bufs[bi]
            d1 = pltpu.async_copy(row_hbm.at[pl.ds(b, CHUNK)], row_v, isem)
            d2 = pltpu.async_copy(col_hbm.at[pl.ds(b, CHUNK)], col_v, isem)
            d3 = pltpu.async_copy(ew_hbm.at[pl.ds(b, CHUNK)], ew_v, isem)
            return (d1, d2, d3)

        def scale(bi):
            ew_v, rows_v = bufs[bi][2], bufs[bi][3]

            def body(g, _):
                w = ew_v[pl.ds(g * 16, 16)]
                for i in range(16):
                    s = w[i]
                    e = g * 16 + i
                    for j in range(HW // 16):
                        sl = pl.ds(j * 16, 16)
                        rows_v[e, sl] = rows_v[e, sl] * s
                return 0
            lax.fori_loop(0, CHUNK // 16, body, 0)

        for half, y_s in ((0, ya_s), (1, yb_s)):
            rows_v0_ = bufs[0][3]

            def zrow(e, _):
                for j in range(HW // 16):
                    rows_v0_[e, pl.ds(j * 16, 16)] = zeros16
                return 0
            lax.fori_loop(0, SUB_ROWS, zrow, 0)
            pltpu.sync_copy(rows_v0_.at[pl.ds(0, SUB_ROWS)],
                            acc.at[pl.ds(sid * SUB_ROWS, SUB_ROWS)])
            plsc.subcore_barrier()

            scatters = [None, None]
            for d in idx_load(0, base0):
                d.wait()
            g0 = pltpu.async_copy(y_s.at[bufs[0][0]], bufs[0][3], bufs[0][4])
            gathers = [g0, None]
            for kk in range(N_CHUNKS):
                A = kk % 2
                Bx = 1 - A
                gathers[A].wait()
                nxt = kk + 1 < N_CHUNKS
                if nxt:
                    idx_descs = idx_load(Bx, base0 + (kk + 1) * CHUNK)
                scale(A)
                if nxt:
                    for d in idx_descs:
                        d.wait()
                    if scatters[Bx] is not None:
                        scatters[Bx].wait()
                    gathers[Bx] = pltpu.async_copy(
                        y_s.at[bufs[Bx][0]], bufs[Bx][3], bufs[Bx][4])
                scatters[A] = pltpu.async_copy(
                    bufs[A][3], acc.at[bufs[A][1]], bufs[A][5], add=True)
            for d in scatters:
                if d is not None:
                    d.wait()
            plsc.subcore_barrier()

            pltpu.sync_copy(
                acc.at[pl.ds(sid * SUB_ROWS, SUB_ROWS)],
                out_hbm.at[pl.ds((cid * 2 + half) * N_PAD + sid * SUB_ROWS,
                                 SUB_ROWS)])

    return k(y_a, y_b, row_p, col_p, ew_p)


def kernel(x_static, edge_index, edge_attr, demand_seq,
           W_gcn1, b_gcn1, W_gcn2, b_gcn2,
           W_ih, W_hh, b_ih, b_hh, W_out, b_out):
    pad = E_PAD - E
    row_p = jnp.concatenate([edge_index[0], jnp.zeros((pad,), jnp.int32)])
    col_p = jnp.concatenate([edge_index[1], jnp.zeros((pad,), jnp.int32)])
    ew_p = jnp.concatenate([edge_attr, jnp.zeros((pad,), jnp.float32)])

    deg = _sc_deg(col_p, ew_p)
    y1a, y1b, dinv = _tc1(x_static, W_gcn1, deg[:N_PAD, None], deg[N_PAD:, None])

    acc1 = _sc_agg(y1a, y1b, row_p, col_p, ew_p)
    y2a, y2b = _tc2(acc1[0:N_PAD], acc1[N_PAD:2 * N_PAD],
                    acc1[2 * N_PAD:3 * N_PAD], acc1[3 * N_PAD:4 * N_PAD],
                    y1a, y1b, dinv, b_gcn1[None], W_gcn2)

    acc2 = _sc_agg(y2a, y2b, row_p, col_p, ew_p)

    dem_bn = jnp.transpose(demand_seq, (0, 2, 1)).reshape(B * N, L)
    pred = _tc3(acc2[0:N_PAD], acc2[N_PAD:2 * N_PAD],
                acc2[2 * N_PAD:3 * N_PAD], acc2[3 * N_PAD:4 * N_PAD],
                y2a, y2b, dinv, b_gcn2[None], dem_bn,
                W_ih.reshape(1, 4 * H), W_hh, (b_ih + b_hh)[None],
                W_out[:, :H], W_out[:, H:], b_out[None])
    return pred.reshape(B, N, OUT)
